# baseline (speedup 1.0000x reference)
import functools

import jax
import jax.numpy as jnp
from jax import lax
from jax.experimental import pallas as pl
from jax.experimental.pallas import tpu as pltpu
from jax.experimental.pallas import tpu_sc as plsc

_TQ = 256
_TN = 2048
_KC = 64
_SC_LANES = 16


def _argmin_body(n_keys, kc, q_ref, k_ref, fprev_ref, idx_ref, psum_ref):
    j = pl.program_id(1)
    qx = jnp.transpose(q_ref[0, :, 0:1])
    qy = jnp.transpose(q_ref[0, :, 1:2])
    tq = qx.shape[1]
    ii = lax.broadcasted_iota(jnp.int32, (kc, tq), 0)
    rmin = jnp.full((kc, tq), jnp.inf, jnp.float32)
    ridx = jnp.zeros((kc, tq), jnp.int32)
    for k in range(n_keys // kc):
        kx = k_ref[0, k * kc:(k + 1) * kc, 0:1]
        ky = k_ref[0, k * kc:(k + 1) * kc, 1:2]
        dc = jnp.abs(kx - qx) + jnp.abs(ky - qy)
        m = dc < rmin
        rmin = jnp.where(m, dc, rmin)
        ridx = jnp.where(m, ii + (k * kc), ridx)
    dmin = jnp.min(rmin, axis=0, keepdims=True)
    sel = jnp.where(rmin == dmin, ridx, n_keys)
    idx_ref[0, 0, :] = jnp.min(sel, axis=0)

    @pl.when(j == 0)
    def _():
        psum_ref[...] = jnp.zeros_like(psum_ref)

    psum_ref[0, :, :] += jnp.sum(fprev_ref[0], axis=0, keepdims=True)


def _nn_half(pos_org, pos_shuffled, feat_prev, off, nb):
    B, N, _ = pos_org.shape
    C = feat_prev.shape[2]
    grid = (nb, N // _TQ)
    idx, psum = pl.pallas_call(
        functools.partial(_argmin_body, N, _KC),
        grid=grid,
        in_specs=[
            pl.BlockSpec((1, _TQ, 2), lambda b, j: (b + off, j, 0)),
            pl.BlockSpec((1, N, 2), lambda b, j: (b + off, 0, 0)),
            pl.BlockSpec((1, _TQ, C), lambda b, j: (b + off, j, 0)),
        ],
        out_specs=[
            pl.BlockSpec((1, 1, _TQ), lambda b, j: (b, 0, j)),
            pl.BlockSpec((1, 1, C), lambda b, j: (b, 0, 0)),
        ],
        out_shape=[
            jax.ShapeDtypeStruct((nb, 1, N), jnp.int32),
            jax.ShapeDtypeStruct((nb, 1, C), jnp.float32),
        ],
        compiler_params=pltpu.CompilerParams(
            dimension_semantics=("parallel", "arbitrary"),
        ),
    )(pos_org, pos_shuffled, feat_prev)
    return idx.reshape(nb, N), psum


def _sc_hist_body(n_bins, n_batches, idx_hbm, cnt_hbm, idx_v, cnt_v):
    wid = lax.axis_index("s") * 2 + lax.axis_index("c")

    @pl.when(wid < n_batches)
    def _():
        pltpu.sync_copy(idx_hbm.at[wid], idx_v)
        zeros = jnp.zeros((_SC_LANES,), jnp.float32)
        ones = jnp.ones((_SC_LANES,), jnp.float32)

        def zero_body(i, carry):
            cnt_v[pl.ds(i * _SC_LANES, _SC_LANES)] = zeros
            return carry

        lax.fori_loop(0, n_bins // _SC_LANES, zero_body, 0)

        def add_body(i, carry):
            iv = idx_v[pl.ds(i * _SC_LANES, _SC_LANES)]
            plsc.addupdate_scatter(cnt_v, [iv], ones)
            return carry

        lax.fori_loop(0, n_bins // _SC_LANES, add_body, 0)
        pltpu.sync_copy(cnt_v, cnt_hbm.at[wid])


def _index_histogram(idx):
    nb, N = idx.shape
    mesh = plsc.VectorSubcoreMesh(core_axis_name="c", subcore_axis_name="s")
    hist = pl.kernel(
        functools.partial(_sc_hist_body, N, nb),
        mesh=mesh,
        out_type=jax.ShapeDtypeStruct((nb, N), jnp.float32),
        scratch_types=[
            pltpu.VMEM((N,), jnp.int32),
            pltpu.VMEM((N,), jnp.float32),
        ],
        compiler_params=pltpu.CompilerParams(needs_layout_passes=False),
    )
    return hist(idx)


def _proj_core(N, feat_ref, w_ref, bp_ref, g_ref, b_ref, cnt_ref):
    x = feat_ref[0]
    proj = jnp.dot(x, w_ref[...], preferred_element_type=jnp.float32)
    proj = proj + bp_ref[0:1, :]
    mu = jnp.mean(proj, axis=1, keepdims=True)
    var = jnp.mean((proj - mu) ** 2, axis=1, keepdims=True)
    ln = (proj - mu) / jnp.sqrt(var + 1e-5) * g_ref[0:1, :] + b_ref[0:1, :]
    c = cnt_ref[0, :, :]
    wsum = jnp.dot(c, ln, preferred_element_type=jnp.float32)
    return wsum * (1.0 / N)


def _proj_partial_body(nb, N, feat_ref, w_ref, bp_ref, g_ref, b_ref, cnt_ref,
                       acc_ref):
    b = pl.program_id(0)
    j = pl.program_id(1)

    @pl.when((b == 0) & (j == 0))
    def _():
        acc_ref[...] = jnp.zeros_like(acc_ref)

    wmean = _proj_core(N, feat_ref, w_ref, bp_ref, g_ref, b_ref, cnt_ref)
    row = lax.broadcasted_iota(jnp.int32, (nb, 1), 0)
    acc_ref[:, 0, :] += jnp.where(row == b, wmean, 0.0)


def _proj_partial(feat, counts, W_proj, b_proj, ln_g, ln_b, off, nb):
    B, N, C = feat.shape
    grid = (nb, N // _TN)
    const = lambda b, j: (0, 0)
    return pl.pallas_call(
        functools.partial(_proj_partial_body, nb, N),
        grid=grid,
        in_specs=[
            pl.BlockSpec((1, _TN, C), lambda b, j: (b + off, j, 0)),
            pl.BlockSpec((C, C), const),
            pl.BlockSpec((1, C), const),
            pl.BlockSpec((1, C), const),
            pl.BlockSpec((1, C), const),
            pl.BlockSpec((1, 1, _TN), lambda b, j: (b, 0, j)),
        ],
        out_specs=pl.BlockSpec((nb, 1, C), lambda b, j: (0, 0, 0)),
        out_shape=jax.ShapeDtypeStruct((nb, 1, C), jnp.float32),
        compiler_params=pltpu.CompilerParams(
            dimension_semantics=("arbitrary", "arbitrary"),
        ),
    )(feat, W_proj, b_proj.reshape(1, C), ln_g.reshape(1, C),
      ln_b.reshape(1, C), counts.reshape(nb, 1, N))


def _proj_final_body(nb, N, C, feat_ref, w_ref, bp_ref, g_ref, b_ref, cnt_ref,
                     accA_ref, psA_ref, psB_ref, w1_ref, b1_ref, w2_ref,
                     b2_ref, w3_ref, b3_ref, o_ref, acc_ref):
    b = pl.program_id(0)
    j = pl.program_id(1)
    nj = pl.num_programs(1)

    @pl.when((b == 0) & (j == 0))
    def _():
        acc_ref[...] = jnp.zeros_like(acc_ref)

    wmean = _proj_core(N, feat_ref, w_ref, bp_ref, g_ref, b_ref, cnt_ref)
    row = lax.broadcasted_iota(jnp.int32, (nb, 1), 0)
    acc_ref[...] += jnp.where(row == b, wmean, 0.0)

    @pl.when((b == nb - 1) & (j == nj - 1))
    def _():
        prevA = psA_ref[:, 0, :] * (1.0 / N)
        prevB = psB_ref[:, 0, :] * (1.0 / N)
        fusedA = accA_ref[:, 0, :] + prevA
        fusedB = acc_ref[...] + prevB
        fused = jnp.concatenate([fusedA, fusedB], axis=0)
        prev = jnp.concatenate([prevA, prevB], axis=0)
        pooled = jnp.concatenate([fused, prev], axis=1)
        h = jnp.dot(pooled, w1_ref[...], preferred_element_type=jnp.float32)
        h = jnp.maximum(h + b1_ref[0:1, :], 0.0)
        h = jnp.dot(h, w2_ref[...], preferred_element_type=jnp.float32)
        h = jnp.maximum(h + b2_ref[0:1, :], 0.0)
        o = jnp.dot(h, w3_ref[...], preferred_element_type=jnp.float32)
        o_ref[...] = o + b3_ref[0:1, :]


def _proj_final(feat, counts, accA, psA, psB, W_proj, b_proj, ln_g, ln_b,
                W1, b1, W2, b2, W3, b3, off, nb):
    B, N, C = feat.shape
    NC = W3.shape[1]
    grid = (nb, N // _TN)
    const = lambda b, j: (0, 0)
    const3 = lambda b, j: (0, 0, 0)
    return pl.pallas_call(
        functools.partial(_proj_final_body, nb, N, C),
        grid=grid,
        in_specs=[
            pl.BlockSpec((1, _TN, C), lambda b, j: (b + off, j, 0)),
            pl.BlockSpec((C, C), const),
            pl.BlockSpec((1, C), const),
            pl.BlockSpec((1, C), const),
            pl.BlockSpec((1, C), const),
            pl.BlockSpec((1, 1, _TN), lambda b, j: (b, 0, j)),
            pl.BlockSpec((nb, 1, C), const3),
            pl.BlockSpec((nb, 1, C), const3),
            pl.BlockSpec((nb, 1, C), const3),
            pl.BlockSpec((2 * C, C), const),
            pl.BlockSpec((1, C), const),
            pl.BlockSpec((C, C), const),
            pl.BlockSpec((1, C), const),
            pl.BlockSpec((C, NC), const),
            pl.BlockSpec((1, NC), const),
        ],
        out_specs=pl.BlockSpec((2 * nb, NC), const),
        out_shape=jax.ShapeDtypeStruct((2 * nb, NC), jnp.float32),
        scratch_shapes=[pltpu.VMEM((nb, C), jnp.float32)],
        compiler_params=pltpu.CompilerParams(
            dimension_semantics=("arbitrary", "arbitrary"),
        ),
    )(feat, W_proj, b_proj.reshape(1, C), ln_g.reshape(1, C),
      ln_b.reshape(1, C), counts.reshape(nb, 1, N), accA, psA, psB,
      W1, b1.reshape(1, C), W2, b2.reshape(1, C), W3, b3.reshape(1, NC))


def kernel(pos_org, pos_shuffled, feat, feat_prev, W_proj, b_proj, ln_g, ln_b,
           W1, b1, W2, b2, W3, b3):
    B = pos_org.shape[0]
    nb = B // 2
    idxA, psA = _nn_half(pos_org, pos_shuffled, feat_prev, 0, nb)
    idxB, psB = _nn_half(pos_org, pos_shuffled, feat_prev, nb, nb)
    cntA = _index_histogram(idxA)
    cntB = _index_histogram(idxB)
    accA = _proj_partial(feat, cntA, W_proj, b_proj, ln_g, ln_b, 0, nb)
    return _proj_final(feat, cntB, accA, psA, psB, W_proj, b_proj, ln_g, ln_b,
                       W1, b1, W2, b2, W3, b3, nb, nb)


# speedup vs baseline: 1.2041x; 1.2041x over previous
import functools

import jax
import jax.numpy as jnp
from jax import lax
from jax.experimental import pallas as pl
from jax.experimental.pallas import tpu as pltpu
from jax.experimental.pallas import tpu_sc as plsc

_TQ = 512
_TN = 2048
_SC_LANES = 16


def _argmin_body(n_keys, kc, q_ref, k_ref, fprev_ref, idx_ref, psum_ref):
    j = pl.program_id(1)
    qx = jnp.transpose(q_ref[0, :, 0:1])
    qy = jnp.transpose(q_ref[0, :, 1:2])
    tq = qx.shape[1]
    ii = lax.broadcasted_iota(jnp.int32, (kc, tq), 0)
    rmin = jnp.full((kc, tq), jnp.inf, jnp.float32)
    ridx = jnp.zeros((kc, tq), jnp.int32)
    for k in range(n_keys // kc):
        kx = k_ref[0, k * kc:(k + 1) * kc, 0:1]
        ky = k_ref[0, k * kc:(k + 1) * kc, 1:2]
        dc = jnp.abs(kx - qx) + jnp.abs(ky - qy)
        m = dc < rmin
        rmin = jnp.where(m, dc, rmin)
        ridx = jnp.where(m, ii + (k * kc), ridx)
    dmin = jnp.min(rmin, axis=0, keepdims=True)
    sel = jnp.where(rmin == dmin, ridx, n_keys)
    idx_ref[0, 0, :] = jnp.min(sel, axis=0)

    @pl.when(j == 0)
    def _():
        psum_ref[...] = jnp.zeros_like(psum_ref)

    psum_ref[0, :, :] += jnp.sum(fprev_ref[0], axis=0, keepdims=True)


def _nn_indices_and_prev_sum(pos_org, pos_shuffled, feat_prev):
    B, N, _ = pos_org.shape
    C = feat_prev.shape[2]
    grid = (B, N // _TQ)
    idx, psum = pl.pallas_call(
        functools.partial(_argmin_body, N, 32),
        grid=grid,
        in_specs=[
            pl.BlockSpec((1, _TQ, 2), lambda b, j: (b, j, 0)),
            pl.BlockSpec((1, N, 2), lambda b, j: (b, 0, 0)),
            pl.BlockSpec((1, _TQ, C), lambda b, j: (b, j, 0)),
        ],
        out_specs=[
            pl.BlockSpec((1, 1, _TQ), lambda b, j: (b, 0, j)),
            pl.BlockSpec((1, 1, C), lambda b, j: (b, 0, 0)),
        ],
        out_shape=[
            jax.ShapeDtypeStruct((B, 1, N), jnp.int32),
            jax.ShapeDtypeStruct((B, 1, C), jnp.float32),
        ],
        compiler_params=pltpu.CompilerParams(
            dimension_semantics=("parallel", "arbitrary"),
        ),
    )(pos_org, pos_shuffled, feat_prev)
    return idx.reshape(B, N), psum


def _sc_hist_body(n_bins, n_batches, idx_hbm, cnt_hbm, idx_v, cnt_v):
    wid = lax.axis_index("s") * 2 + lax.axis_index("c")

    @pl.when(wid < n_batches)
    def _():
        pltpu.sync_copy(idx_hbm.at[wid], idx_v)
        zeros = jnp.zeros((_SC_LANES,), jnp.float32)
        ones = jnp.ones((_SC_LANES,), jnp.float32)

        def zero_body(i, carry):
            cnt_v[pl.ds(i * _SC_LANES, _SC_LANES)] = zeros
            return carry

        lax.fori_loop(0, n_bins // _SC_LANES, zero_body, 0)

        def add_body(i, carry):
            iv = idx_v[pl.ds(i * _SC_LANES, _SC_LANES)]
            plsc.addupdate_scatter(cnt_v, [iv], ones)
            return carry

        lax.fori_loop(0, n_bins // _SC_LANES, add_body, 0)
        pltpu.sync_copy(cnt_v, cnt_hbm.at[wid])


def _index_histogram(idx):
    B, N = idx.shape
    mesh = plsc.VectorSubcoreMesh(core_axis_name="c", subcore_axis_name="s")
    hist = pl.kernel(
        functools.partial(_sc_hist_body, N, B),
        mesh=mesh,
        out_type=jax.ShapeDtypeStruct((B, N), jnp.float32),
        scratch_types=[
            pltpu.VMEM((N,), jnp.int32),
            pltpu.VMEM((N,), jnp.float32),
        ],
        compiler_params=pltpu.CompilerParams(needs_layout_passes=False),
    )
    return hist(idx)


def _proj_mlp_body(B, N, C, feat_ref, w_ref, bp_ref, g_ref, b_ref,
                   cnt_ref, psum_ref, w1_ref, b1_ref, w2_ref, b2_ref,
                   w3_ref, b3_ref, o_ref, acc_ref):
    b = pl.program_id(0)
    j = pl.program_id(1)
    nj = pl.num_programs(1)

    @pl.when((b == 0) & (j == 0))
    def _():
        acc_ref[...] = jnp.zeros_like(acc_ref)

    x = feat_ref[0]
    proj = jnp.dot(x, w_ref[...], preferred_element_type=jnp.float32)
    proj = proj + bp_ref[0:1, :]
    mu = jnp.mean(proj, axis=1, keepdims=True)
    var = jnp.mean((proj - mu) ** 2, axis=1, keepdims=True)
    ln = (proj - mu) / jnp.sqrt(var + 1e-5) * g_ref[0:1, :] + b_ref[0:1, :]
    c = cnt_ref[0, :, :]
    wsum = jnp.dot(c, ln, preferred_element_type=jnp.float32)
    row = lax.broadcasted_iota(jnp.int32, (B, 1), 0)
    acc_ref[...] += jnp.where(row == b, wsum * (1.0 / N), 0.0)

    @pl.when((b == B - 1) & (j == nj - 1))
    def _():
        prev_mean = psum_ref[:, 0, :] * (1.0 / N)
        fused_mean = acc_ref[...] + prev_mean
        pooled = jnp.concatenate([fused_mean, prev_mean], axis=1)
        h = jnp.dot(pooled, w1_ref[...], preferred_element_type=jnp.float32)
        h = jnp.maximum(h + b1_ref[0:1, :], 0.0)
        h = jnp.dot(h, w2_ref[...], preferred_element_type=jnp.float32)
        h = jnp.maximum(h + b2_ref[0:1, :], 0.0)
        o = jnp.dot(h, w3_ref[...], preferred_element_type=jnp.float32)
        o_ref[...] = o + b3_ref[0:1, :]


def _proj_mlp(feat, counts, prev_sums, W_proj, b_proj, ln_g, ln_b,
              W1, b1, W2, b2, W3, b3):
    B, N, C = feat.shape
    NC = W3.shape[1]
    grid = (B, N // _TN)
    const = lambda b, j: (0, 0)
    const3 = lambda b, j: (0, 0, 0)
    return pl.pallas_call(
        functools.partial(_proj_mlp_body, B, N, C),
        grid=grid,
        in_specs=[
            pl.BlockSpec((1, _TN, C), lambda b, j: (b, j, 0)),
            pl.BlockSpec((C, C), const),
            pl.BlockSpec((1, C), const),
            pl.BlockSpec((1, C), const),
            pl.BlockSpec((1, C), const),
            pl.BlockSpec((1, 1, _TN), lambda b, j: (b, 0, j)),
            pl.BlockSpec((B, 1, C), const3),
            pl.BlockSpec((2 * C, C), const),
            pl.BlockSpec((1, C), const),
            pl.BlockSpec((C, C), const),
            pl.BlockSpec((1, C), const),
            pl.BlockSpec((C, NC), const),
            pl.BlockSpec((1, NC), const),
        ],
        out_specs=pl.BlockSpec((B, NC), const),
        out_shape=jax.ShapeDtypeStruct((B, NC), jnp.float32),
        scratch_shapes=[pltpu.VMEM((B, C), jnp.float32)],
        compiler_params=pltpu.CompilerParams(
            dimension_semantics=("arbitrary", "arbitrary"),
        ),
    )(feat, W_proj, b_proj.reshape(1, C), ln_g.reshape(1, C),
      ln_b.reshape(1, C), counts.reshape(B, 1, N), prev_sums,
      W1, b1.reshape(1, C), W2, b2.reshape(1, C), W3, b3.reshape(1, NC))


def kernel(pos_org, pos_shuffled, feat, feat_prev, W_proj, b_proj, ln_g, ln_b,
           W1, b1, W2, b2, W3, b3):
    idx, prev_sums = _nn_indices_and_prev_sum(pos_org, pos_shuffled, feat_prev)
    counts = _index_histogram(idx)
    return _proj_mlp(feat, counts, prev_sums, W_proj, b_proj, ln_g, ln_b,
                     W1, b1, W2, b2, W3, b3)


# speedup vs baseline: 1.2239x; 1.0164x over previous
import functools

import jax
import jax.numpy as jnp
from jax import lax
from jax.experimental import pallas as pl
from jax.experimental.pallas import tpu as pltpu
from jax.experimental.pallas import tpu_sc as plsc

_TQ = 2048
_TN = 2048
_SC_LANES = 16


def _argmin_body(n_keys, kc, q_ref, k_ref, fprev_ref, idx_ref, psum_ref):
    j = pl.program_id(1)
    qx = jnp.transpose(q_ref[0, :, 0:1])
    qy = jnp.transpose(q_ref[0, :, 1:2])
    tq = qx.shape[1]
    ii = lax.broadcasted_iota(jnp.int32, (kc, tq), 0)
    rmin = jnp.full((kc, tq), jnp.inf, jnp.float32)
    ridx = jnp.zeros((kc, tq), jnp.int32)
    for k in range(n_keys // kc):
        kx = k_ref[0, k * kc:(k + 1) * kc, 0:1]
        ky = k_ref[0, k * kc:(k + 1) * kc, 1:2]
        dc = jnp.abs(kx - qx) + jnp.abs(ky - qy)
        m = dc < rmin
        rmin = jnp.where(m, dc, rmin)
        ridx = jnp.where(m, ii + (k * kc), ridx)
    dmin = jnp.min(rmin, axis=0, keepdims=True)
    sel = jnp.where(rmin == dmin, ridx, n_keys)
    idx_ref[0, 0, :] = jnp.min(sel, axis=0)

    @pl.when(j == 0)
    def _():
        psum_ref[...] = jnp.zeros_like(psum_ref)

    psum_ref[0, :, :] += jnp.sum(fprev_ref[0], axis=0, keepdims=True)


def _nn_indices_and_prev_sum(pos_org, pos_shuffled, feat_prev):
    B, N, _ = pos_org.shape
    C = feat_prev.shape[2]
    grid = (B, N // _TQ)
    idx, psum = pl.pallas_call(
        functools.partial(_argmin_body, N, 8),
        grid=grid,
        in_specs=[
            pl.BlockSpec((1, _TQ, 2), lambda b, j: (b, j, 0)),
            pl.BlockSpec((1, N, 2), lambda b, j: (b, 0, 0)),
            pl.BlockSpec((1, _TQ, C), lambda b, j: (b, j, 0)),
        ],
        out_specs=[
            pl.BlockSpec((1, 1, _TQ), lambda b, j: (b, 0, j)),
            pl.BlockSpec((1, 1, C), lambda b, j: (b, 0, 0)),
        ],
        out_shape=[
            jax.ShapeDtypeStruct((B, 1, N), jnp.int32),
            jax.ShapeDtypeStruct((B, 1, C), jnp.float32),
        ],
        compiler_params=pltpu.CompilerParams(
            dimension_semantics=("parallel", "arbitrary"),
        ),
    )(pos_org, pos_shuffled, feat_prev)
    return idx.reshape(B, N), psum


def _sc_hist_body(n_bins, n_batches, idx_hbm, cnt_hbm, idx_v, cnt_v):
    wid = lax.axis_index("s") * 2 + lax.axis_index("c")

    @pl.when(wid < n_batches)
    def _():
        pltpu.sync_copy(idx_hbm.at[wid], idx_v)
        zeros = jnp.zeros((_SC_LANES,), jnp.float32)
        ones = jnp.ones((_SC_LANES,), jnp.float32)

        def zero_body(i, carry):
            cnt_v[pl.ds(i * _SC_LANES, _SC_LANES)] = zeros
            return carry

        lax.fori_loop(0, n_bins // _SC_LANES, zero_body, 0)

        def add_body(i, carry):
            iv = idx_v[pl.ds(i * _SC_LANES, _SC_LANES)]
            plsc.addupdate_scatter(cnt_v, [iv], ones)
            return carry

        lax.fori_loop(0, n_bins // _SC_LANES, add_body, 0)
        pltpu.sync_copy(cnt_v, cnt_hbm.at[wid])


def _index_histogram(idx):
    B, N = idx.shape
    mesh = plsc.VectorSubcoreMesh(core_axis_name="c", subcore_axis_name="s")
    hist = pl.kernel(
        functools.partial(_sc_hist_body, N, B),
        mesh=mesh,
        out_type=jax.ShapeDtypeStruct((B, N), jnp.float32),
        scratch_types=[
            pltpu.VMEM((N,), jnp.int32),
            pltpu.VMEM((N,), jnp.float32),
        ],
        compiler_params=pltpu.CompilerParams(needs_layout_passes=False),
    )
    return hist(idx)


def _proj_mlp_body(B, N, C, feat_ref, w_ref, bp_ref, g_ref, b_ref,
                   cnt_ref, psum_ref, w1_ref, b1_ref, w2_ref, b2_ref,
                   w3_ref, b3_ref, o_ref, acc_ref):
    b = pl.program_id(0)
    j = pl.program_id(1)
    nj = pl.num_programs(1)

    @pl.when((b == 0) & (j == 0))
    def _():
        acc_ref[...] = jnp.zeros_like(acc_ref)

    x = feat_ref[0]
    proj = jnp.dot(x, w_ref[...], preferred_element_type=jnp.float32)
    proj = proj + bp_ref[0:1, :]
    mu = jnp.mean(proj, axis=1, keepdims=True)
    var = jnp.mean((proj - mu) ** 2, axis=1, keepdims=True)
    ln = (proj - mu) / jnp.sqrt(var + 1e-5) * g_ref[0:1, :] + b_ref[0:1, :]
    c = cnt_ref[0, :, :]
    wsum = jnp.dot(c, ln, preferred_element_type=jnp.float32)
    row = lax.broadcasted_iota(jnp.int32, (B, 1), 0)
    acc_ref[...] += jnp.where(row == b, wsum * (1.0 / N), 0.0)

    @pl.when((b == B - 1) & (j == nj - 1))
    def _():
        prev_mean = psum_ref[:, 0, :] * (1.0 / N)
        fused_mean = acc_ref[...] + prev_mean
        pooled = jnp.concatenate([fused_mean, prev_mean], axis=1)
        h = jnp.dot(pooled, w1_ref[...], preferred_element_type=jnp.float32)
        h = jnp.maximum(h + b1_ref[0:1, :], 0.0)
        h = jnp.dot(h, w2_ref[...], preferred_element_type=jnp.float32)
        h = jnp.maximum(h + b2_ref[0:1, :], 0.0)
        o = jnp.dot(h, w3_ref[...], preferred_element_type=jnp.float32)
        o_ref[...] = o + b3_ref[0:1, :]


def _proj_mlp(feat, counts, prev_sums, W_proj, b_proj, ln_g, ln_b,
              W1, b1, W2, b2, W3, b3):
    B, N, C = feat.shape
    NC = W3.shape[1]
    grid = (B, N // _TN)
    const = lambda b, j: (0, 0)
    const3 = lambda b, j: (0, 0, 0)
    return pl.pallas_call(
        functools.partial(_proj_mlp_body, B, N, C),
        grid=grid,
        in_specs=[
            pl.BlockSpec((1, _TN, C), lambda b, j: (b, j, 0)),
            pl.BlockSpec((C, C), const),
            pl.BlockSpec((1, C), const),
            pl.BlockSpec((1, C), const),
            pl.BlockSpec((1, C), const),
            pl.BlockSpec((1, 1, _TN), lambda b, j: (b, 0, j)),
            pl.BlockSpec((B, 1, C), const3),
            pl.BlockSpec((2 * C, C), const),
            pl.BlockSpec((1, C), const),
            pl.BlockSpec((C, C), const),
            pl.BlockSpec((1, C), const),
            pl.BlockSpec((C, NC), const),
            pl.BlockSpec((1, NC), const),
        ],
        out_specs=pl.BlockSpec((B, NC), const),
        out_shape=jax.ShapeDtypeStruct((B, NC), jnp.float32),
        scratch_shapes=[pltpu.VMEM((B, C), jnp.float32)],
        compiler_params=pltpu.CompilerParams(
            dimension_semantics=("arbitrary", "arbitrary"),
        ),
    )(feat, W_proj, b_proj.reshape(1, C), ln_g.reshape(1, C),
      ln_b.reshape(1, C), counts.reshape(B, 1, N), prev_sums,
      W1, b1.reshape(1, C), W2, b2.reshape(1, C), W3, b3.reshape(1, NC))


def kernel(pos_org, pos_shuffled, feat, feat_prev, W_proj, b_proj, ln_g, ln_b,
           W1, b1, W2, b2, W3, b3):
    idx, prev_sums = _nn_indices_and_prev_sum(pos_org, pos_shuffled, feat_prev)
    counts = _index_histogram(idx)
    return _proj_mlp(feat, counts, prev_sums, W_proj, b_proj, ln_g, ln_b,
                     W1, b1, W2, b2, W3, b3)


# speedup vs baseline: 1.2353x; 1.0093x over previous
import functools

import jax
import jax.numpy as jnp
from jax import lax
from jax.experimental import pallas as pl
from jax.experimental.pallas import tpu as pltpu
from jax.experimental.pallas import tpu_sc as plsc

_TQ = 1024
_TN = 2048
_SC_LANES = 16


def _argmin_body(n_keys, kc, q_ref, k_ref, fprev_ref, idx_ref, psum_ref):
    j = pl.program_id(1)
    qx = jnp.transpose(q_ref[0, :, 0:1])
    qy = jnp.transpose(q_ref[0, :, 1:2])
    tq = qx.shape[1]
    ii = lax.broadcasted_iota(jnp.int32, (kc, tq), 0)
    rmin = jnp.full((kc, tq), jnp.inf, jnp.float32)
    ridx = jnp.zeros((kc, tq), jnp.int32)
    for k in range(n_keys // kc):
        kx = k_ref[0, k * kc:(k + 1) * kc, 0:1]
        ky = k_ref[0, k * kc:(k + 1) * kc, 1:2]
        dc = jnp.abs(kx - qx) + jnp.abs(ky - qy)
        m = dc < rmin
        rmin = jnp.where(m, dc, rmin)
        ridx = jnp.where(m, ii + (k * kc), ridx)
    dmin = jnp.min(rmin, axis=0, keepdims=True)
    sel = jnp.where(rmin == dmin, ridx, n_keys)
    idx_ref[0, 0, :] = jnp.min(sel, axis=0)

    @pl.when(j == 0)
    def _():
        psum_ref[...] = jnp.zeros_like(psum_ref)

    psum_ref[0, :, :] += jnp.sum(fprev_ref[0], axis=0, keepdims=True)


def _nn_indices_and_prev_sum(pos_org, pos_shuffled, feat_prev):
    B, N, _ = pos_org.shape
    C = feat_prev.shape[2]
    grid = (B, N // _TQ)
    idx, psum = pl.pallas_call(
        functools.partial(_argmin_body, N, 16),
        grid=grid,
        in_specs=[
            pl.BlockSpec((1, _TQ, 2), lambda b, j: (b, j, 0)),
            pl.BlockSpec((1, N, 2), lambda b, j: (b, 0, 0)),
            pl.BlockSpec((1, _TQ, C), lambda b, j: (b, j, 0)),
        ],
        out_specs=[
            pl.BlockSpec((1, 1, _TQ), lambda b, j: (b, 0, j)),
            pl.BlockSpec((1, 1, C), lambda b, j: (b, 0, 0)),
        ],
        out_shape=[
            jax.ShapeDtypeStruct((B, 1, N), jnp.int32),
            jax.ShapeDtypeStruct((B, 1, C), jnp.float32),
        ],
        compiler_params=pltpu.CompilerParams(
            dimension_semantics=("parallel", "arbitrary"),
        ),
    )(pos_org, pos_shuffled, feat_prev)
    return idx.reshape(B, N), psum


def _sc_hist_body(n_bins, n_batches, idx_hbm, cnt_hbm, idx_v, cnt_v):
    wid = lax.axis_index("s") * 2 + lax.axis_index("c")

    @pl.when(wid < n_batches)
    def _():
        pltpu.sync_copy(idx_hbm.at[wid], idx_v)
        zeros = jnp.zeros((_SC_LANES,), jnp.float32)
        ones = jnp.ones((_SC_LANES,), jnp.float32)

        def zero_body(i, carry):
            cnt_v[pl.ds(i * _SC_LANES, _SC_LANES)] = zeros
            return carry

        lax.fori_loop(0, n_bins // _SC_LANES, zero_body, 0)

        def add_body(i, carry):
            iv = idx_v[pl.ds(i * _SC_LANES, _SC_LANES)]
            plsc.addupdate_scatter(cnt_v, [iv], ones)
            return carry

        lax.fori_loop(0, n_bins // _SC_LANES, add_body, 0)
        pltpu.sync_copy(cnt_v, cnt_hbm.at[wid])


def _index_histogram(idx):
    B, N = idx.shape
    mesh = plsc.VectorSubcoreMesh(core_axis_name="c", subcore_axis_name="s")
    hist = pl.kernel(
        functools.partial(_sc_hist_body, N, B),
        mesh=mesh,
        out_type=jax.ShapeDtypeStruct((B, N), jnp.float32),
        scratch_types=[
            pltpu.VMEM((N,), jnp.int32),
            pltpu.VMEM((N,), jnp.float32),
        ],
        compiler_params=pltpu.CompilerParams(needs_layout_passes=False),
    )
    return hist(idx)


def _proj_mlp_body(B, N, C, feat_ref, w_ref, bp_ref, g_ref, b_ref,
                   cnt_ref, psum_ref, w1_ref, b1_ref, w2_ref, b2_ref,
                   w3_ref, b3_ref, o_ref, acc_ref):
    b = pl.program_id(0)
    j = pl.program_id(1)
    nj = pl.num_programs(1)

    @pl.when((b == 0) & (j == 0))
    def _():
        acc_ref[...] = jnp.zeros_like(acc_ref)

    x = feat_ref[0]
    proj = jnp.dot(x, w_ref[...], preferred_element_type=jnp.float32)
    proj = proj + bp_ref[0:1, :]
    mu = jnp.mean(proj, axis=1, keepdims=True)
    var = jnp.mean((proj - mu) ** 2, axis=1, keepdims=True)
    ln = (proj - mu) / jnp.sqrt(var + 1e-5) * g_ref[0:1, :] + b_ref[0:1, :]
    c = cnt_ref[0, :, :]
    wsum = jnp.dot(c, ln, preferred_element_type=jnp.float32)
    row = lax.broadcasted_iota(jnp.int32, (B, 1), 0)
    acc_ref[...] += jnp.where(row == b, wsum * (1.0 / N), 0.0)

    @pl.when((b == B - 1) & (j == nj - 1))
    def _():
        prev_mean = psum_ref[:, 0, :] * (1.0 / N)
        fused_mean = acc_ref[...] + prev_mean
        pooled = jnp.concatenate([fused_mean, prev_mean], axis=1)
        h = jnp.dot(pooled, w1_ref[...], preferred_element_type=jnp.float32)
        h = jnp.maximum(h + b1_ref[0:1, :], 0.0)
        h = jnp.dot(h, w2_ref[...], preferred_element_type=jnp.float32)
        h = jnp.maximum(h + b2_ref[0:1, :], 0.0)
        o = jnp.dot(h, w3_ref[...], preferred_element_type=jnp.float32)
        o_ref[...] = o + b3_ref[0:1, :]


def _proj_mlp(feat, counts, prev_sums, W_proj, b_proj, ln_g, ln_b,
              W1, b1, W2, b2, W3, b3):
    B, N, C = feat.shape
    NC = W3.shape[1]
    grid = (B, N // _TN)
    const = lambda b, j: (0, 0)
    const3 = lambda b, j: (0, 0, 0)
    return pl.pallas_call(
        functools.partial(_proj_mlp_body, B, N, C),
        grid=grid,
        in_specs=[
            pl.BlockSpec((1, _TN, C), lambda b, j: (b, j, 0)),
            pl.BlockSpec((C, C), const),
            pl.BlockSpec((1, C), const),
            pl.BlockSpec((1, C), const),
            pl.BlockSpec((1, C), const),
            pl.BlockSpec((1, 1, _TN), lambda b, j: (b, 0, j)),
            pl.BlockSpec((B, 1, C), const3),
            pl.BlockSpec((2 * C, C), const),
            pl.BlockSpec((1, C), const),
            pl.BlockSpec((C, C), const),
            pl.BlockSpec((1, C), const),
            pl.BlockSpec((C, NC), const),
            pl.BlockSpec((1, NC), const),
        ],
        out_specs=pl.BlockSpec((B, NC), const),
        out_shape=jax.ShapeDtypeStruct((B, NC), jnp.float32),
        scratch_shapes=[pltpu.VMEM((B, C), jnp.float32)],
        compiler_params=pltpu.CompilerParams(
            dimension_semantics=("arbitrary", "arbitrary"),
        ),
    )(feat, W_proj, b_proj.reshape(1, C), ln_g.reshape(1, C),
      ln_b.reshape(1, C), counts.reshape(B, 1, N), prev_sums,
      W1, b1.reshape(1, C), W2, b2.reshape(1, C), W3, b3.reshape(1, NC))


def kernel(pos_org, pos_shuffled, feat, feat_prev, W_proj, b_proj, ln_g, ln_b,
           W1, b1, W2, b2, W3, b3):
    idx, prev_sums = _nn_indices_and_prev_sum(pos_org, pos_shuffled, feat_prev)
    counts = _index_histogram(idx)
    return _proj_mlp(feat, counts, prev_sums, W_proj, b_proj, ln_g, ln_b,
                     W1, b1, W2, b2, W3, b3)
